# baseline (speedup 1.0000x reference)
import functools

import jax
import jax.numpy as jnp
from jax.experimental import pallas as pl
from jax.experimental.pallas import tpu as pltpu


def _layer1_body(x_ref, adj_ref, w_ref, b_ref, h1_ref, adj16_ref, z_ref, *, zc):
    m = pl.program_id(0)
    n = z_ref.shape[0]

    @pl.when(m == 0)
    def _():
        def body(i, carry):
            sl = pl.ds(i * zc, zc)
            z_ref[sl, :] = jax.lax.dot_general(
                x_ref[sl, :], w_ref[...], (((1,), (0,)), ((), ())),
                precision=jax.lax.Precision.HIGHEST,
                preferred_element_type=jnp.float32).astype(jnp.bfloat16)
            return carry
        jax.lax.fori_loop(0, n // zc, body, 0)

    a16 = adj_ref[...].astype(jnp.bfloat16)
    adj16_ref[...] = a16
    h = jax.lax.dot_general(
        a16, z_ref[...], (((1,), (0,)), ((), ())),
        preferred_element_type=jnp.float32)
    h1_ref[...] = jnp.maximum(h + b_ref[...], 0.0)


def _layers23_body(h1_ref, adj16_ref, w_ref, b_ref, out_ref, z_ref, h_ref,
                   *, bm, zc):
    l = pl.program_id(0)
    m = pl.program_id(1)
    n = h_ref.shape[0]

    def _project(src_ref):
        def body(i, carry):
            sl = pl.ds(i * zc, zc)
            z_ref[sl, :] = jax.lax.dot_general(
                src_ref[sl, :], w_ref[0], (((1,), (0,)), ((), ())),
                precision=jax.lax.Precision.HIGHEST,
                preferred_element_type=jnp.float32).astype(jnp.bfloat16)
            return carry
        jax.lax.fori_loop(0, n // zc, body, 0)

    @pl.when(jnp.logical_and(m == 0, l == 0))
    def _():
        _project(h1_ref)

    @pl.when(jnp.logical_and(m == 0, l == 1))
    def _():
        _project(h_ref)

    h = jax.lax.dot_general(
        adj16_ref[...], z_ref[...], (((1,), (0,)), ((), ())),
        preferred_element_type=jnp.float32)
    h = jnp.maximum(h + b_ref[0], 0.0)

    @pl.when(l == 0)
    def _():
        h_ref[pl.ds(m * bm, bm), :] = h

    @pl.when(l == 1)
    def _():
        mx = jnp.max(h, axis=1, keepdims=True)
        s = jnp.sum(jnp.exp(h - mx), axis=1, keepdims=True)
        out_ref[...] = h - mx - jnp.log(s)


def kernel(x, adj, W1, b1, W2, b2, W3, b3):
    n, d = x.shape
    bm = 200 if n % 200 == 0 else n
    nm = n // bm
    zc = 1000 if n % 1000 == 0 else n

    h1, adj16 = pl.pallas_call(
        functools.partial(_layer1_body, zc=zc),
        grid=(nm,),
        in_specs=[
            pl.BlockSpec((n, d), lambda m: (0, 0)),
            pl.BlockSpec((bm, n), lambda m: (m, 0)),
            pl.BlockSpec((d, d), lambda m: (0, 0)),
            pl.BlockSpec((1, d), lambda m: (0, 0)),
        ],
        out_specs=[
            pl.BlockSpec((bm, d), lambda m: (m, 0)),
            pl.BlockSpec((bm, n), lambda m: (m, 0)),
        ],
        out_shape=[
            jax.ShapeDtypeStruct((n, d), jnp.float32),
            jax.ShapeDtypeStruct((n, n), jnp.bfloat16),
        ],
        scratch_shapes=[
            pltpu.VMEM((n, d), jnp.bfloat16),
        ],
        compiler_params=pltpu.CompilerParams(
            dimension_semantics=("arbitrary",),
            vmem_limit_bytes=100 * 1024 * 1024,
        ),
    )(x, adj, W1, b1.reshape(1, d))

    w = jnp.stack([W2, W3])
    b = jnp.stack([b2, b3]).reshape(2, 1, d)

    return pl.pallas_call(
        functools.partial(_layers23_body, bm=bm, zc=zc),
        grid=(2, nm),
        in_specs=[
            pl.BlockSpec((n, d), lambda l, m: (0, 0)),
            pl.BlockSpec((bm, n), lambda l, m: (m, 0)),
            pl.BlockSpec((1, d, d), lambda l, m: (l, 0, 0)),
            pl.BlockSpec((1, 1, d), lambda l, m: (l, 0, 0)),
        ],
        out_specs=pl.BlockSpec(
            (bm, d), lambda l, m: (jnp.where(l == 1, m, 0), 0)),
        out_shape=jax.ShapeDtypeStruct((n, d), jnp.float32),
        scratch_shapes=[
            pltpu.VMEM((n, d), jnp.bfloat16),
            pltpu.VMEM((n, d), jnp.float32),
        ],
        compiler_params=pltpu.CompilerParams(
            dimension_semantics=("arbitrary", "arbitrary"),
            vmem_limit_bytes=100 * 1024 * 1024,
        ),
    )(h1, adj16, w, b)


# speedup vs baseline: 1.0577x; 1.0577x over previous
import functools

import jax
import jax.numpy as jnp
from jax.experimental import pallas as pl
from jax.experimental.pallas import tpu as pltpu


def _layer1_body(x_ref, adj_ref, w_ref, b_ref, h1_ref, adj16_ref, z_ref, *, zc):
    m = pl.program_id(0)
    n = z_ref.shape[0]

    @pl.when(m == 0)
    def _():
        def body(i, carry):
            sl = pl.ds(i * zc, zc)
            z_ref[sl, :] = jax.lax.dot_general(
                x_ref[sl, :], w_ref[...], (((1,), (0,)), ((), ())),
                preferred_element_type=jnp.float32).astype(jnp.bfloat16)
            return carry
        jax.lax.fori_loop(0, n // zc, body, 0)

    a16 = adj_ref[...].astype(jnp.bfloat16)
    adj16_ref[...] = a16
    h = jax.lax.dot_general(
        a16, z_ref[...], (((1,), (0,)), ((), ())),
        preferred_element_type=jnp.float32)
    h1_ref[...] = jnp.maximum(h + b_ref[...], 0.0)


def _layers23_body(h1_ref, adj16_ref, w_ref, b_ref, out_ref, z_ref, h_ref,
                   *, bm, zc):
    l = pl.program_id(0)
    m = pl.program_id(1)
    n = h_ref.shape[0]

    def _project(src_ref):
        def body(i, carry):
            sl = pl.ds(i * zc, zc)
            z_ref[sl, :] = jax.lax.dot_general(
                src_ref[sl, :], w_ref[0], (((1,), (0,)), ((), ())),
                preferred_element_type=jnp.float32).astype(jnp.bfloat16)
            return carry
        jax.lax.fori_loop(0, n // zc, body, 0)

    @pl.when(jnp.logical_and(m == 0, l == 0))
    def _():
        _project(h1_ref)

    @pl.when(jnp.logical_and(m == 0, l == 1))
    def _():
        _project(h_ref)

    h = jax.lax.dot_general(
        adj16_ref[...], z_ref[...], (((1,), (0,)), ((), ())),
        preferred_element_type=jnp.float32)
    h = jnp.maximum(h + b_ref[0], 0.0)

    @pl.when(l == 0)
    def _():
        h_ref[pl.ds(m * bm, bm), :] = h

    @pl.when(l == 1)
    def _():
        mx = jnp.max(h, axis=1, keepdims=True)
        s = jnp.sum(jnp.exp(h - mx), axis=1, keepdims=True)
        out_ref[...] = h - mx - jnp.log(s)


def kernel(x, adj, W1, b1, W2, b2, W3, b3):
    n, d = x.shape
    bm = 200 if n % 200 == 0 else n
    nm = n // bm
    zc = 2000 if n % 2000 == 0 else n

    h1, adj16 = pl.pallas_call(
        functools.partial(_layer1_body, zc=zc),
        grid=(nm,),
        in_specs=[
            pl.BlockSpec((n, d), lambda m: (0, 0)),
            pl.BlockSpec((bm, n), lambda m: (m, 0)),
            pl.BlockSpec((d, d), lambda m: (0, 0)),
            pl.BlockSpec((1, d), lambda m: (0, 0)),
        ],
        out_specs=[
            pl.BlockSpec((bm, d), lambda m: (m, 0)),
            pl.BlockSpec((bm, n), lambda m: (m, 0)),
        ],
        out_shape=[
            jax.ShapeDtypeStruct((n, d), jnp.float32),
            jax.ShapeDtypeStruct((n, n), jnp.bfloat16),
        ],
        scratch_shapes=[
            pltpu.VMEM((n, d), jnp.bfloat16),
        ],
        compiler_params=pltpu.CompilerParams(
            dimension_semantics=("arbitrary",),
            vmem_limit_bytes=100 * 1024 * 1024,
        ),
    )(x, adj, W1, b1.reshape(1, d))

    w = jnp.stack([W2, W3])
    b = jnp.stack([b2, b3]).reshape(2, 1, d)

    return pl.pallas_call(
        functools.partial(_layers23_body, bm=bm, zc=zc),
        grid=(2, nm),
        in_specs=[
            pl.BlockSpec((n, d), lambda l, m: (0, 0)),
            pl.BlockSpec((bm, n), lambda l, m: (m, 0)),
            pl.BlockSpec((1, d, d), lambda l, m: (l, 0, 0)),
            pl.BlockSpec((1, 1, d), lambda l, m: (l, 0, 0)),
        ],
        out_specs=pl.BlockSpec(
            (bm, d), lambda l, m: (jnp.where(l == 1, m, 0), 0)),
        out_shape=jax.ShapeDtypeStruct((n, d), jnp.float32),
        scratch_shapes=[
            pltpu.VMEM((n, d), jnp.bfloat16),
            pltpu.VMEM((n, d), jnp.float32),
        ],
        compiler_params=pltpu.CompilerParams(
            dimension_semantics=("arbitrary", "arbitrary"),
            vmem_limit_bytes=100 * 1024 * 1024,
        ),
    )(h1, adj16, w, b)


# speedup vs baseline: 1.1499x; 1.0871x over previous
import functools

import jax
import jax.numpy as jnp
from jax.experimental import pallas as pl
from jax.experimental.pallas import tpu as pltpu


def _layer1_body(x_ref, adj_ref, w1_ref, b1_ref, w2_ref,
                 z2_ref, adj16_ref, z1_ref, *, zc):
    m = pl.program_id(0)
    n = z1_ref.shape[0]

    @pl.when(m == 0)
    def _():
        def body(i, carry):
            sl = pl.ds(i * zc, zc)
            z1_ref[sl, :] = jax.lax.dot_general(
                x_ref[sl, :], w1_ref[...], (((1,), (0,)), ((), ())),
                preferred_element_type=jnp.float32).astype(jnp.bfloat16)
            return carry
        jax.lax.fori_loop(0, n // zc, body, 0)

    a16 = adj_ref[...].astype(jnp.bfloat16)
    adj16_ref[...] = a16
    h1 = jax.lax.dot_general(
        a16, z1_ref[...], (((1,), (0,)), ((), ())),
        preferred_element_type=jnp.float32)
    h1 = jnp.maximum(h1 + b1_ref[...], 0.0)
    z2_ref[...] = jax.lax.dot_general(
        h1, w2_ref[...], (((1,), (0,)), ((), ())),
        preferred_element_type=jnp.float32).astype(jnp.bfloat16)


def _layers23_body(z2_ref, adj16_ref, w3_ref, b_ref, out_ref, z3_ref, h_ref,
                   *, bm, zc):
    l = pl.program_id(0)
    m = pl.program_id(1)
    n = h_ref.shape[0]

    @pl.when(jnp.logical_and(m == 0, l == 1))
    def _():
        def body(i, carry):
            sl = pl.ds(i * zc, zc)
            z3_ref[sl, :] = jax.lax.dot_general(
                h_ref[sl, :], w3_ref[...], (((1,), (0,)), ((), ())),
                preferred_element_type=jnp.float32).astype(jnp.bfloat16)
            return carry
        jax.lax.fori_loop(0, n // zc, body, 0)

    a16 = adj16_ref[...]

    @pl.when(l == 0)
    def _():
        h = jax.lax.dot_general(a16, z2_ref[...], (((1,), (0,)), ((), ())),
                                preferred_element_type=jnp.float32)
        h = jnp.maximum(h + b_ref[0], 0.0)
        h_ref[pl.ds(m * bm, bm), :] = h.astype(jnp.bfloat16)

    @pl.when(l == 1)
    def _():
        h = jax.lax.dot_general(a16, z3_ref[...], (((1,), (0,)), ((), ())),
                                preferred_element_type=jnp.float32)
        h = jnp.maximum(h + b_ref[1], 0.0)
        mx = jnp.max(h, axis=1, keepdims=True)
        s = jnp.sum(jnp.exp(h - mx), axis=1, keepdims=True)
        out_ref[...] = h - mx - jnp.log(s)


def kernel(x, adj, W1, b1, W2, b2, W3, b3):
    n, d = x.shape
    bma = 200 if n % 200 == 0 else n
    nma = n // bma
    bmb = 400 if n % 400 == 0 else n
    nmb = n // bmb
    zc = 2000 if n % 2000 == 0 else n

    z2, adj16 = pl.pallas_call(
        functools.partial(_layer1_body, zc=zc),
        grid=(nma,),
        in_specs=[
            pl.BlockSpec((n, d), lambda m: (0, 0)),
            pl.BlockSpec((bma, n), lambda m: (m, 0)),
            pl.BlockSpec((d, d), lambda m: (0, 0)),
            pl.BlockSpec((1, d), lambda m: (0, 0)),
            pl.BlockSpec((d, d), lambda m: (0, 0)),
        ],
        out_specs=[
            pl.BlockSpec((bma, d), lambda m: (m, 0)),
            pl.BlockSpec((bma, n), lambda m: (m, 0)),
        ],
        out_shape=[
            jax.ShapeDtypeStruct((n, d), jnp.bfloat16),
            jax.ShapeDtypeStruct((n, n), jnp.bfloat16),
        ],
        scratch_shapes=[
            pltpu.VMEM((n, d), jnp.bfloat16),
        ],
        compiler_params=pltpu.CompilerParams(
            dimension_semantics=("arbitrary",),
            vmem_limit_bytes=100 * 1024 * 1024,
        ),
    )(x, adj, W1, b1.reshape(1, d), W2)

    b23 = jnp.stack([b2, b3]).reshape(2, 1, d)

    return pl.pallas_call(
        functools.partial(_layers23_body, bm=bmb, zc=zc),
        grid=(2, nmb),
        in_specs=[
            pl.BlockSpec((n, d), lambda l, m: (0, 0)),
            pl.BlockSpec((bmb, n), lambda l, m: (m, 0)),
            pl.BlockSpec((d, d), lambda l, m: (0, 0)),
            pl.BlockSpec((2, 1, d), lambda l, m: (0, 0, 0)),
        ],
        out_specs=pl.BlockSpec(
            (bmb, d), lambda l, m: (jnp.where(l == 1, m, 0), 0)),
        out_shape=jax.ShapeDtypeStruct((n, d), jnp.float32),
        scratch_shapes=[
            pltpu.VMEM((n, d), jnp.bfloat16),
            pltpu.VMEM((n, d), jnp.bfloat16),
        ],
        compiler_params=pltpu.CompilerParams(
            dimension_semantics=("arbitrary", "arbitrary"),
            vmem_limit_bytes=100 * 1024 * 1024,
        ),
    )(z2, adj16, W3, b23)


# speedup vs baseline: 1.1873x; 1.0325x over previous
import functools

import jax
import jax.numpy as jnp
from jax.experimental import pallas as pl
from jax.experimental.pallas import tpu as pltpu


def _layer1_body(x_ref, adj_ref, w1_ref, b1_ref, w2_ref,
                 z2_ref, adj16_ref, z1_ref, *, zc, npad):
    m = pl.program_id(0)
    n = z1_ref.shape[0]

    @pl.when(m == 0)
    def _():
        def body(i, carry):
            sl = pl.ds(i * zc, zc)
            z1_ref[sl, :] = jax.lax.dot_general(
                x_ref[sl, :], w1_ref[...], (((1,), (0,)), ((), ())),
                preferred_element_type=jnp.float32).astype(jnp.bfloat16)
            return carry
        jax.lax.fori_loop(0, n // zc, body, 0)

    a16 = adj_ref[...].astype(jnp.bfloat16)
    bma = a16.shape[0]
    adj16_ref[...] = jnp.concatenate(
        [a16, jnp.zeros((bma, npad - n), jnp.bfloat16)], axis=1)
    h1 = jax.lax.dot_general(
        a16, z1_ref[...], (((1,), (0,)), ((), ())),
        preferred_element_type=jnp.float32)
    h1 = jnp.maximum(h1 + b1_ref[...], 0.0)
    z2_ref[...] = jax.lax.dot_general(
        h1, w2_ref[...], (((1,), (0,)), ((), ())),
        preferred_element_type=jnp.float32).astype(jnp.bfloat16)


def _layers23_body(z2_ref, adj16_ref, w3_ref, b_ref, out_ref,
                   zp_ref, h2_ref, acc_ref, *, bm, bk, nk, zc):
    l = pl.program_id(0)
    m = pl.program_id(1)
    k = pl.program_id(2)
    n = h2_ref.shape[0]
    npad = zp_ref.shape[0]

    @pl.when(jnp.logical_and(l == 0, jnp.logical_and(m == 0, k == 0)))
    def _():
        zp_ref[pl.ds(0, n), :] = z2_ref[...]
        zp_ref[pl.ds(n, npad - n), :] = jnp.zeros((npad - n, zp_ref.shape[1]),
                                                  jnp.bfloat16)

    @pl.when(jnp.logical_and(l == 1, jnp.logical_and(m == 0, k == 0)))
    def _():
        def body(i, carry):
            sl = pl.ds(i * zc, zc)
            zp_ref[sl, :] = jax.lax.dot_general(
                h2_ref[sl, :], w3_ref[...], (((1,), (0,)), ((), ())),
                preferred_element_type=jnp.float32).astype(jnp.bfloat16)
            return carry
        jax.lax.fori_loop(0, n // zc, body, 0)

    part = jax.lax.dot_general(
        adj16_ref[...], zp_ref[pl.ds(k * bk, bk), :], (((1,), (0,)), ((), ())),
        preferred_element_type=jnp.float32)

    @pl.when(k == 0)
    def _():
        acc_ref[...] = part

    @pl.when(k > 0)
    def _():
        acc_ref[...] += part

    @pl.when(k == nk - 1)
    def _():
        @pl.when(l == 0)
        def _():
            h = jnp.maximum(acc_ref[...] + b_ref[0], 0.0)
            h2_ref[pl.ds(m * bm, bm), :] = h

        @pl.when(l == 1)
        def _():
            h = jnp.maximum(acc_ref[...] + b_ref[1], 0.0)
            mx = jnp.max(h, axis=1, keepdims=True)
            s = jnp.sum(jnp.exp(h - mx), axis=1, keepdims=True)
            out_ref[...] = h - mx - jnp.log(s)


def kernel(x, adj, W1, b1, W2, b2, W3, b3):
    n, d = x.shape
    bma = 200 if n % 200 == 0 else n
    nma = n // bma
    bmb = 1000 if n % 1000 == 0 else n
    nmb = n // bmb
    zc = 2000 if n % 2000 == 0 else n
    npad = -(-n // 2048) * 2048
    nk = 2
    bk = npad // nk

    z2, adj16 = pl.pallas_call(
        functools.partial(_layer1_body, zc=zc, npad=npad),
        grid=(nma,),
        in_specs=[
            pl.BlockSpec((n, d), lambda m: (0, 0)),
            pl.BlockSpec((bma, n), lambda m: (m, 0)),
            pl.BlockSpec((d, d), lambda m: (0, 0)),
            pl.BlockSpec((1, d), lambda m: (0, 0)),
            pl.BlockSpec((d, d), lambda m: (0, 0)),
        ],
        out_specs=[
            pl.BlockSpec((bma, d), lambda m: (m, 0)),
            pl.BlockSpec((bma, npad), lambda m: (m, 0)),
        ],
        out_shape=[
            jax.ShapeDtypeStruct((n, d), jnp.bfloat16),
            jax.ShapeDtypeStruct((n, npad), jnp.bfloat16),
        ],
        scratch_shapes=[
            pltpu.VMEM((n, d), jnp.bfloat16),
        ],
        compiler_params=pltpu.CompilerParams(
            dimension_semantics=("arbitrary",),
            vmem_limit_bytes=100 * 1024 * 1024,
        ),
    )(x, adj, W1, b1.reshape(1, d), W2)

    b23 = jnp.stack([b2, b3]).reshape(2, 1, d)

    return pl.pallas_call(
        functools.partial(_layers23_body, bm=bmb, bk=bk, nk=nk, zc=zc),
        grid=(2, nmb, nk),
        in_specs=[
            pl.BlockSpec((n, d), lambda l, m, k: (0, 0)),
            pl.BlockSpec((bmb, bk), lambda l, m, k: (m, k)),
            pl.BlockSpec((d, d), lambda l, m, k: (0, 0)),
            pl.BlockSpec((2, 1, d), lambda l, m, k: (0, 0, 0)),
        ],
        out_specs=pl.BlockSpec(
            (bmb, d), lambda l, m, k: (jnp.where(l == 1, m, 0), 0)),
        out_shape=jax.ShapeDtypeStruct((n, d), jnp.float32),
        scratch_shapes=[
            pltpu.VMEM((npad, d), jnp.bfloat16),
            pltpu.VMEM((n, d), jnp.float32),
            pltpu.VMEM((bmb, d), jnp.float32),
        ],
        compiler_params=pltpu.CompilerParams(
            dimension_semantics=("arbitrary", "arbitrary", "arbitrary"),
            vmem_limit_bytes=100 * 1024 * 1024,
        ),
    )(z2, adj16, W3, b23)
